# Initial kernel scaffold; baseline (speedup 1.0000x reference)
import jax
import jax.numpy as jnp
from jax.experimental import pallas as pl


def kernel(x, pos, edge_index, edge_attr, batch, conv1_W, conv1_b, pool1_W, pool1_b, conv2_W, conv2_b, pool2_W, pool2_b, conv3_W, conv3_b, pool3_W, pool3_b, lin1_W, lin1_b, lin2_W, lin2_b, lin3_W, lin3_b):
    raise NotImplementedError("write your pallas kernel here")


# speedup vs baseline: 19.8612x; 19.8612x over previous
import functools

import jax
import jax.numpy as jnp
from jax import lax
from jax.experimental import pallas as pl
from jax.experimental.pallas import tpu as pltpu
from jax.experimental.pallas import tpu_sc as plsc

N = 10000
NP = 10240
E = 320000
D = 128
NC = 2
NS = 16
NW = NC * NS
EPT = 10240
EP = NW * EPT
CCH = 256
SCH = 2048
RPT = NP // NS

_f32 = jnp.float32
_i32 = jnp.int32


def _mesh():
    return plsc.VectorSubcoreMesh(core_axis_name="c", subcore_axis_name="s")


_sc_params = pltpu.CompilerParams(needs_layout_passes=False)


def _edge_prep_body(src_hbm, dst_hbm, ew_hbm, alive_hbm, ewo_hbm, degp_hbm,
                    alive_v, deg_v, src_v, dst_v, ew_v, ewo_v):
    c = lax.axis_index("c")
    s = lax.axis_index("s")
    wid = c * NS + s
    pltpu.sync_copy(alive_hbm, alive_v)
    zeros16 = jnp.zeros((16,), _f32)

    def zbody(i, carry):
        deg_v[pl.ds(pl.multiple_of(i * 16, 16), 16)] = zeros16
        return carry
    lax.fori_loop(0, NP // 16, zbody, 0)

    def cbody(ci, carry):
        off = wid * EPT + ci * SCH
        pltpu.sync_copy(src_hbm.at[pl.ds(off, SCH)], src_v)
        pltpu.sync_copy(dst_hbm.at[pl.ds(off, SCH)], dst_v)
        pltpu.sync_copy(ew_hbm.at[pl.ds(off, SCH)], ew_v)

        def ibody(j, icarry):
            sl = pl.ds(pl.multiple_of(j * 16, 16), 16)
            s16 = src_v[sl]
            d16 = dst_v[sl]
            w16 = ew_v[sl]
            a_s = plsc.load_gather(alive_v, [s16])
            a_d = plsc.load_gather(alive_v, [d16])
            wv = w16 * a_s * a_d
            ewo_v[sl] = wv
            plsc.addupdate_scatter(deg_v, [d16], wv)
            return icarry
        lax.fori_loop(0, SCH // 16, ibody, 0)
        pltpu.sync_copy(ewo_v, ewo_hbm.at[pl.ds(off, SCH)])
        return carry
    lax.fori_loop(0, EPT // SCH, cbody, 0)
    pltpu.sync_copy(deg_v, degp_hbm.at[pl.ds(wid * NP, NP)])


def _edge_prep_call(src, dst, ew, alive):
    out_type = (jax.ShapeDtypeStruct((EP,), _f32),
                jax.ShapeDtypeStruct((NW * NP,), _f32))
    scratch = [pltpu.VMEM((N,), _f32), pltpu.VMEM((NP,), _f32),
               pltpu.VMEM((SCH,), _i32), pltpu.VMEM((SCH,), _i32),
               pltpu.VMEM((SCH,), _f32), pltpu.VMEM((SCH,), _f32)]
    fn = pl.kernel(_edge_prep_body, out_type=out_type, mesh=_mesh(),
                   scratch_types=scratch, name="edge_prep",
                   compiler_params=_sc_params)
    return fn(src, dst, ew, alive)


def _conv_body(hp_hbm, src_hbm, dst_hbm, ew_hbm, out_hbm,
               acc_sh, src_v, dst_v, ew_v, rows_v, sem):
    c = lax.axis_index("c")
    s = lax.axis_index("s")
    wid = c * NS + s
    zeros16 = jnp.zeros((16,), _f32)

    def zr(r, carry):
        for j in range(8):
            rows_v[r, pl.ds(j * 16, 16)] = zeros16
        return carry
    lax.fori_loop(0, CCH, zr, 0)

    base = s * RPT
    pltpu.sync_copy(rows_v, acc_sh.at[pl.ds(base, CCH)])
    pltpu.sync_copy(rows_v, acc_sh.at[pl.ds(base + CCH, CCH)])
    pltpu.sync_copy(rows_v.at[pl.ds(0, RPT - 2 * CCH)],
                    acc_sh.at[pl.ds(base + 2 * CCH, RPT - 2 * CCH)])
    plsc.subcore_barrier()

    def cbody(ci, carry):
        off = wid * EPT + ci * CCH
        pltpu.sync_copy(src_hbm.at[pl.ds(off, CCH)], src_v)
        pltpu.sync_copy(dst_hbm.at[pl.ds(off, CCH)], dst_v)
        pltpu.sync_copy(ew_hbm.at[pl.ds(off, CCH)], ew_v)
        pltpu.async_copy(hp_hbm.at[src_v], rows_v, sem).wait()

        def sbody(jj, icarry):
            w16 = ew_v[pl.ds(pl.multiple_of(jj * 16, 16), 16)]
            for t in range(16):
                w = w16[t]
                r = jj * 16 + t
                for j in range(8):
                    sl = pl.ds(j * 16, 16)
                    rows_v[r, sl] = rows_v[r, sl] * w
            return icarry
        lax.fori_loop(0, CCH // 16, sbody, 0)
        pltpu.sync_copy(rows_v, acc_sh.at[dst_v], add=True)
        return carry
    lax.fori_loop(0, EPT // CCH, cbody, 0)
    plsc.subcore_barrier()
    pltpu.sync_copy(acc_sh.at[pl.ds(s * RPT, RPT)],
                    out_hbm.at[pl.ds(c * NP + s * RPT, RPT)])


def _conv_call(hp, src, dst, ew):
    out_type = jax.ShapeDtypeStruct((NC * NP, D), _f32)
    scratch = [pltpu.VMEM_SHARED((NP, D), _f32),
               pltpu.VMEM((CCH,), _i32), pltpu.VMEM((CCH,), _i32),
               pltpu.VMEM((CCH,), _f32), pltpu.VMEM((CCH, D), _f32),
               pltpu.SemaphoreType.DMA]
    fn = pl.kernel(_conv_body, out_type=out_type, mesh=_mesh(),
                   scratch_types=scratch, name="conv_pass",
                   compiler_params=_sc_params)
    return fn(hp, src, dst, ew)


def _score_body(sp_hbm, src_hbm, dst_hbm, ew_hbm, saccp_hbm,
                sp_v, sacc_v, src_v, dst_v, ew_v):
    c = lax.axis_index("c")
    s = lax.axis_index("s")
    wid = c * NS + s
    pltpu.sync_copy(sp_hbm, sp_v)
    zeros16 = jnp.zeros((16,), _f32)

    def zbody(i, carry):
        sacc_v[pl.ds(pl.multiple_of(i * 16, 16), 16)] = zeros16
        return carry
    lax.fori_loop(0, NP // 16, zbody, 0)

    def cbody(ci, carry):
        off = wid * EPT + ci * SCH
        pltpu.sync_copy(src_hbm.at[pl.ds(off, SCH)], src_v)
        pltpu.sync_copy(dst_hbm.at[pl.ds(off, SCH)], dst_v)
        pltpu.sync_copy(ew_hbm.at[pl.ds(off, SCH)], ew_v)

        def ibody(j, icarry):
            sl = pl.ds(pl.multiple_of(j * 16, 16), 16)
            s16 = src_v[sl]
            d16 = dst_v[sl]
            w16 = ew_v[sl]
            v = plsc.load_gather(sp_v, [s16]) * w16
            plsc.addupdate_scatter(sacc_v, [d16], v)
            return icarry
        lax.fori_loop(0, SCH // 16, ibody, 0)
        return carry
    lax.fori_loop(0, EPT // SCH, cbody, 0)
    pltpu.sync_copy(sacc_v, saccp_hbm.at[pl.ds(wid * NP, NP)])


def _score_call(sp, src, dst, ew):
    out_type = jax.ShapeDtypeStruct((NW * NP,), _f32)
    scratch = [pltpu.VMEM((N,), _f32), pltpu.VMEM((NP,), _f32),
               pltpu.VMEM((SCH,), _i32), pltpu.VMEM((SCH,), _i32),
               pltpu.VMEM((SCH,), _f32)]
    fn = pl.kernel(_score_body, out_type=out_type, mesh=_mesh(),
                   scratch_types=scratch, name="score_pass",
                   compiler_params=_sc_params)
    return fn(sp, src, dst, ew)


def _comb_deg_body(degp_ref, dinv_ref):
    deg = 1.0 + jnp.sum(degp_ref[...], axis=0)
    dinv_ref[...] = lax.rsqrt(deg)


def _comb_deg_call(degp):
    out_shape = jax.ShapeDtypeStruct((NP,), _f32)
    return pl.pallas_call(_comb_deg_body, out_shape=out_shape)(degp)


def _tcA_body(h_ref, wc_ref, dinv_ref, H_ref, hp_ref):
    H = jnp.dot(h_ref[...], wc_ref[...], preferred_element_type=_f32)
    H_ref[...] = H
    hp_ref[...] = H * dinv_ref[...]


def _tcA_call(h, wc, dinv_col):
    out_shape = (jax.ShapeDtypeStruct((N, D), _f32),
                 jax.ShapeDtypeStruct((N, D), _f32))
    return pl.pallas_call(_tcA_body, out_shape=out_shape)(h, wc, dinv_col)


def _tcB_body(accp_ref, H_ref, dinv_ref, bc_ref, ws_ref,
              H2_ref, sp_ref, s_ref):
    acc = accp_ref[0] + accp_ref[1]
    dinv = dinv_ref[...]
    out = dinv * acc + (dinv * dinv) * H_ref[...] + bc_ref[...][None, :]
    H2 = jnp.maximum(out, 0.0)
    s = jnp.dot(H2, ws_ref[...], preferred_element_type=_f32)
    H2_ref[...] = H2
    s_ref[...] = s
    sp_ref[...] = s * dinv


def _tcB_call(accp, H, dinv_col, bc, ws):
    out_shape = (jax.ShapeDtypeStruct((N, D), _f32),
                 jax.ShapeDtypeStruct((N, 1), _f32),
                 jax.ShapeDtypeStruct((N, 1), _f32))
    return pl.pallas_call(_tcB_body, out_shape=out_shape)(
        accp, H, dinv_col, bc, ws)


def _score_top_body(k, saccp_ref, s_ref, dinv_ref, alive_ref, bs_ref,
                    scsel_ref, selv_ref):
    sacc = jnp.sum(saccp_ref[...], axis=0)[:N]
    dinv = dinv_ref[...][:N]
    pre = dinv * sacc + (dinv * dinv) * s_ref[...] + bs_ref[...]
    score = jnp.tanh(pre)
    alive = alive_ref[...]
    masked = jnp.where(alive > 0.0, score, -2.5)
    bits = lax.bitcast_convert_type(masked, _i32)
    key = jnp.where(bits < 0, bits ^ 0x7FFFFFFF, bits)

    def bis(_, lohi):
        lo, hi = lohi
        ulo = lo.astype(jnp.uint32)
        uhi = hi.astype(jnp.uint32)
        mid = (ulo + ((uhi - ulo) >> 1)).astype(_i32)
        cnt = jnp.sum((key > mid).astype(_i32))
        ge = cnt >= k
        return (jnp.where(ge, mid, lo), jnp.where(ge, hi, mid))
    lo0 = jnp.asarray(-(2 ** 31), _i32)
    hi0 = jnp.asarray(2 ** 31 - 1, _i32)
    lo, hi = lax.fori_loop(0, 32, bis, (lo0, hi0))

    c_hi = jnp.sum((key > hi).astype(_i32))
    need = k - c_hi
    tie = key == hi
    idx = lax.broadcasted_iota(_i32, (N,), 0)

    def bis2(_, lohi):
        lo2, hi2 = lohi
        mid = lo2 + (hi2 - lo2) // 2
        cnt = jnp.sum((tie & (idx <= mid)).astype(_i32))
        ge = cnt >= need
        return (jnp.where(ge, lo2, mid), jnp.where(ge, hi2, mid))
    lo2, hi2 = lax.fori_loop(0, 14, bis2,
                             (jnp.asarray(-1, _i32), jnp.asarray(N - 1, _i32)))

    sel = (key > hi) | (tie & (idx <= hi2) & (need > 0))
    selv = sel.astype(_f32)
    scsel_ref[...] = score * selv
    selv_ref[...] = selv


def _score_top_call(k, saccp, s_row, dinv_row, alive, bs):
    out_shape = (jax.ShapeDtypeStruct((N,), _f32),
                 jax.ShapeDtypeStruct((N,), _f32))
    body = functools.partial(_score_top_body, k)
    return pl.pallas_call(body, out_shape=out_shape)(
        saccp, s_row, dinv_row, alive, bs)


def _pool_body(k, H2_ref, scsel_ref, sel_ref, hn_ref, xr_ref):
    hn = H2_ref[...] * scsel_ref[...]
    hn_ref[...] = hn
    mx = jnp.max(jnp.where(sel_ref[...] > 0.0, hn, -3.0e38),
                 axis=0, keepdims=True)
    sm = jnp.sum(hn, axis=0, keepdims=True)
    xr_ref[...] = jnp.concatenate([mx, sm * (1.0 / k)], axis=1)


def _pool_call(k, H2, scsel_col, sel_col):
    out_shape = (jax.ShapeDtypeStruct((N, D), _f32),
                 jax.ShapeDtypeStruct((1, 2 * D), _f32))
    body = functools.partial(_pool_body, k)
    return pl.pallas_call(body, out_shape=out_shape)(H2, scsel_col, sel_col)


def _mlp_body(x1_ref, x2_ref, x3_ref, l1w_ref, l1b_ref, l2w_ref, l2b_ref,
              l3w_ref, l3b_ref, out_ref):
    z = x1_ref[...] + x2_ref[...] + x3_ref[...]
    z = jnp.maximum(
        jnp.dot(z, l1w_ref[...], preferred_element_type=_f32)
        + l1b_ref[...][None, :], 0.0)
    z = jnp.maximum(
        jnp.dot(z, l2w_ref[...], preferred_element_type=_f32)
        + l2b_ref[...][None, :], 0.0)
    z = jnp.dot(z, l3w_ref[...], preferred_element_type=_f32) \
        + l3b_ref[...][None, :]
    out_ref[...] = jax.nn.sigmoid(z)


def _mlp_call(x1, x2, x3, l1w, l1b, l2w, l2b, l3w, l3b):
    out_shape = jax.ShapeDtypeStruct((1, 1), _f32)
    return pl.pallas_call(_mlp_body, out_shape=out_shape)(
        x1, x2, x3, l1w, l1b, l2w, l2b, l3w, l3b)


def kernel(x, pos, edge_index, edge_attr, batch,
           conv1_W, conv1_b, pool1_W, pool1_b,
           conv2_W, conv2_b, pool2_W, pool2_b,
           conv3_W, conv3_b, pool3_W, pool3_b,
           lin1_W, lin1_b, lin2_W, lin2_b, lin3_W, lin3_b):
    src = jnp.zeros((EP,), _i32).at[:E].set(edge_index[0])
    dst = jnp.zeros((EP,), _i32).at[:E].set(edge_index[1])
    ew0 = jnp.zeros((EP,), _f32).at[:E].set(edge_attr)
    h = jnp.concatenate([x, pos], axis=1)
    alive = jnp.ones((N,), _f32)
    layers = ((conv1_W, conv1_b, pool1_W, pool1_b, 5000),
              (conv2_W, conv2_b, pool2_W, pool2_b, 2500),
              (conv3_W, conv3_b, pool3_W, pool3_b, 1250))
    xrs = []
    for (wc, bc, ws, bs, k) in layers:
        ew_i, degp = _edge_prep_call(src, dst, ew0, alive)
        dinv_row = _comb_deg_call(degp.reshape(NW, NP))
        dinv_col = dinv_row[:N].reshape(N, 1)
        H, hp = _tcA_call(h, wc, dinv_col)
        accp = _conv_call(hp, src, dst, ew_i).reshape(NC, NP, D)[:, :N]
        H2, sp, s = _tcB_call(accp, H, dinv_col, bc, ws)
        saccp = _score_call(sp.reshape(N), src, dst, ew_i)
        scsel, selv = _score_top_call(k, saccp.reshape(NW, NP),
                                      s.reshape(N), dinv_row, alive, bs)
        alive = selv
        h, xr = _pool_call(k, H2, scsel.reshape(N, 1), selv.reshape(N, 1))
        xrs.append(xr)
    return _mlp_call(xrs[0], xrs[1], xrs[2],
                     lin1_W, lin1_b, lin2_W, lin2_b, lin3_W, lin3_b)


# speedup vs baseline: 20.5813x; 1.0363x over previous
import functools

import jax
import jax.numpy as jnp
from jax import lax
from jax.experimental import pallas as pl
from jax.experimental.pallas import tpu as pltpu
from jax.experimental.pallas import tpu_sc as plsc

N = 10000
NP = 10240
E = 320000
D = 128
NC = 2
NS = 16
NW = NC * NS
EPT = 10240
EP = NW * EPT
CCH = 128
SCH = 2048
RPT = NP // NS

_f32 = jnp.float32
_i32 = jnp.int32


def _mesh():
    return plsc.VectorSubcoreMesh(core_axis_name="c", subcore_axis_name="s")


_sc_params = pltpu.CompilerParams(needs_layout_passes=False)


def _edge_prep_body(src_hbm, dst_hbm, ew_hbm, alive_hbm, ewo_hbm, degp_hbm,
                    alive_v, deg_v, src_v, dst_v, ew_v, ewo_v):
    c = lax.axis_index("c")
    s = lax.axis_index("s")
    wid = c * NS + s
    pltpu.sync_copy(alive_hbm, alive_v)
    zeros16 = jnp.zeros((16,), _f32)

    def zbody(i, carry):
        deg_v[pl.ds(pl.multiple_of(i * 16, 16), 16)] = zeros16
        return carry
    lax.fori_loop(0, NP // 16, zbody, 0)

    def cbody(ci, carry):
        off = wid * EPT + ci * SCH
        pltpu.sync_copy(src_hbm.at[pl.ds(off, SCH)], src_v)
        pltpu.sync_copy(dst_hbm.at[pl.ds(off, SCH)], dst_v)
        pltpu.sync_copy(ew_hbm.at[pl.ds(off, SCH)], ew_v)

        def ibody(j, icarry):
            sl = pl.ds(pl.multiple_of(j * 16, 16), 16)
            s16 = src_v[sl]
            d16 = dst_v[sl]
            w16 = ew_v[sl]
            a_s = plsc.load_gather(alive_v, [s16])
            a_d = plsc.load_gather(alive_v, [d16])
            wv = w16 * a_s * a_d
            ewo_v[sl] = wv
            plsc.addupdate_scatter(deg_v, [d16], wv)
            return icarry
        lax.fori_loop(0, SCH // 16, ibody, 0)
        pltpu.sync_copy(ewo_v, ewo_hbm.at[pl.ds(off, SCH)])
        return carry
    lax.fori_loop(0, EPT // SCH, cbody, 0)
    pltpu.sync_copy(deg_v, degp_hbm.at[pl.ds(wid * NP, NP)])


def _edge_prep_call(src, dst, ew, alive):
    out_type = (jax.ShapeDtypeStruct((EP,), _f32),
                jax.ShapeDtypeStruct((NW * NP,), _f32))
    scratch = [pltpu.VMEM((N,), _f32), pltpu.VMEM((NP,), _f32),
               pltpu.VMEM((SCH,), _i32), pltpu.VMEM((SCH,), _i32),
               pltpu.VMEM((SCH,), _f32), pltpu.VMEM((SCH,), _f32)]
    fn = pl.kernel(_edge_prep_body, out_type=out_type, mesh=_mesh(),
                   scratch_types=scratch, name="edge_prep",
                   compiler_params=_sc_params)
    return fn(src, dst, ew, alive)


def _conv_body(hp_hbm, src_hbm, dst_hbm, ew_hbm, out_hbm,
               acc_sh, src0, dst0, ew0_v, rows0, src1, dst1, ew1_v, rows1,
               sg0, sg1, ss0, ss1):
    c = lax.axis_index("c")
    s = lax.axis_index("s")
    wid = c * NS + s
    zeros16 = jnp.zeros((16,), _f32)
    srcb = (src0, src1)
    dstb = (dst0, dst1)
    ewb = (ew0_v, ew1_v)
    rowsb = (rows0, rows1)
    sgb = (sg0, sg1)
    ssb = (ss0, ss1)

    def zr(r, carry):
        for j in range(8):
            rows0[r, pl.ds(j * 16, 16)] = zeros16
        return carry
    lax.fori_loop(0, CCH, zr, 0)

    base = s * RPT
    for i in range(RPT // CCH):
        pltpu.sync_copy(rows0, acc_sh.at[pl.ds(base + i * CCH, CCH)])
    plsc.subcore_barrier()

    ebase = wid * EPT
    nch = EPT // CCH

    def prefetch(ci, b):
        off = ebase + ci * CCH
        pltpu.sync_copy(src_hbm.at[pl.ds(off, CCH)], srcb[b])
        pltpu.sync_copy(dst_hbm.at[pl.ds(off, CCH)], dstb[b])
        pltpu.sync_copy(ew_hbm.at[pl.ds(off, CCH)], ewb[b])
        pltpu.async_copy(hp_hbm.at[srcb[b]], rowsb[b], sgb[b])

    prefetch(0, 0)

    def cbody(ci, carry):
        for b in (0, 1):

            @pl.when((ci % 2) == b)
            def _():
                rows_v = rowsb[b]
                ew_v = ewb[b]
                pltpu.make_async_copy(hp_hbm.at[srcb[b]], rows_v,
                                      sgb[b]).wait()

                @pl.when(ci + 1 < nch)
                def _():
                    nb = 1 - b

                    @pl.when(ci >= 1)
                    def _():
                        pltpu.make_async_copy(
                            rowsb[nb], acc_sh.at[dstb[nb]], ssb[nb]).wait()
                    prefetch(ci + 1, nb)

                def sbody(jj, icarry):
                    w16 = ew_v[pl.ds(pl.multiple_of(jj * 16, 16), 16)]
                    for t in range(16):
                        w = w16[t]
                        r = jj * 16 + t
                        for j in range(8):
                            sl = pl.ds(j * 16, 16)
                            rows_v[r, sl] = rows_v[r, sl] * w
                    return icarry
                lax.fori_loop(0, CCH // 16, sbody, 0)
                pltpu.async_copy(rows_v, acc_sh.at[dstb[b]], ssb[b],
                                 add=True)
        return carry
    lax.fori_loop(0, nch, cbody, 0)
    pltpu.make_async_copy(rows0, acc_sh.at[dst0], ss0).wait()
    pltpu.make_async_copy(rows1, acc_sh.at[dst1], ss1).wait()
    plsc.subcore_barrier()
    pltpu.sync_copy(acc_sh.at[pl.ds(s * RPT, RPT)],
                    out_hbm.at[pl.ds(c * NP + s * RPT, RPT)])


def _conv_call(hp, src, dst, ew):
    out_type = jax.ShapeDtypeStruct((NC * NP, D), _f32)
    buf = [pltpu.VMEM((CCH,), _i32), pltpu.VMEM((CCH,), _i32),
           pltpu.VMEM((CCH,), _f32), pltpu.VMEM((CCH, D), _f32)]
    scratch = [pltpu.VMEM_SHARED((NP, D), _f32)] + buf + buf + \
        [pltpu.SemaphoreType.DMA] * 4
    fn = pl.kernel(_conv_body, out_type=out_type, mesh=_mesh(),
                   scratch_types=scratch, name="conv_pass",
                   compiler_params=_sc_params)
    return fn(hp, src, dst, ew)


def _score_body(sp_hbm, src_hbm, dst_hbm, ew_hbm, saccp_hbm,
                sp_v, sacc_v, src_v, dst_v, ew_v):
    c = lax.axis_index("c")
    s = lax.axis_index("s")
    wid = c * NS + s
    pltpu.sync_copy(sp_hbm, sp_v)
    zeros16 = jnp.zeros((16,), _f32)

    def zbody(i, carry):
        sacc_v[pl.ds(pl.multiple_of(i * 16, 16), 16)] = zeros16
        return carry
    lax.fori_loop(0, NP // 16, zbody, 0)

    def cbody(ci, carry):
        off = wid * EPT + ci * SCH
        pltpu.sync_copy(src_hbm.at[pl.ds(off, SCH)], src_v)
        pltpu.sync_copy(dst_hbm.at[pl.ds(off, SCH)], dst_v)
        pltpu.sync_copy(ew_hbm.at[pl.ds(off, SCH)], ew_v)

        def ibody(j, icarry):
            sl = pl.ds(pl.multiple_of(j * 16, 16), 16)
            s16 = src_v[sl]
            d16 = dst_v[sl]
            w16 = ew_v[sl]
            v = plsc.load_gather(sp_v, [s16]) * w16
            plsc.addupdate_scatter(sacc_v, [d16], v)
            return icarry
        lax.fori_loop(0, SCH // 16, ibody, 0)
        return carry
    lax.fori_loop(0, EPT // SCH, cbody, 0)
    pltpu.sync_copy(sacc_v, saccp_hbm.at[pl.ds(wid * NP, NP)])


def _score_call(sp, src, dst, ew):
    out_type = jax.ShapeDtypeStruct((NW * NP,), _f32)
    scratch = [pltpu.VMEM((N,), _f32), pltpu.VMEM((NP,), _f32),
               pltpu.VMEM((SCH,), _i32), pltpu.VMEM((SCH,), _i32),
               pltpu.VMEM((SCH,), _f32)]
    fn = pl.kernel(_score_body, out_type=out_type, mesh=_mesh(),
                   scratch_types=scratch, name="score_pass",
                   compiler_params=_sc_params)
    return fn(sp, src, dst, ew)


def _comb_deg_body(degp_ref, dinv_ref):
    deg = 1.0 + jnp.sum(degp_ref[...], axis=0)
    dinv_ref[...] = lax.rsqrt(deg)


def _comb_deg_call(degp):
    out_shape = jax.ShapeDtypeStruct((NP,), _f32)
    return pl.pallas_call(_comb_deg_body, out_shape=out_shape)(degp)


def _tcA_body(h_ref, wc_ref, dinv_ref, H_ref, hp_ref):
    H = jnp.dot(h_ref[...], wc_ref[...], preferred_element_type=_f32)
    H_ref[...] = H
    hp_ref[...] = H * dinv_ref[...]


def _tcA_call(h, wc, dinv_col):
    out_shape = (jax.ShapeDtypeStruct((N, D), _f32),
                 jax.ShapeDtypeStruct((N, D), _f32))
    return pl.pallas_call(_tcA_body, out_shape=out_shape)(h, wc, dinv_col)


def _tcB_body(accp_ref, H_ref, dinv_ref, bc_ref, ws_ref,
              H2_ref, sp_ref, s_ref):
    acc = accp_ref[0] + accp_ref[1]
    dinv = dinv_ref[...]
    out = dinv * acc + (dinv * dinv) * H_ref[...] + bc_ref[...][None, :]
    H2 = jnp.maximum(out, 0.0)
    s = jnp.dot(H2, ws_ref[...], preferred_element_type=_f32)
    H2_ref[...] = H2
    s_ref[...] = s
    sp_ref[...] = s * dinv


def _tcB_call(accp, H, dinv_col, bc, ws):
    out_shape = (jax.ShapeDtypeStruct((N, D), _f32),
                 jax.ShapeDtypeStruct((N, 1), _f32),
                 jax.ShapeDtypeStruct((N, 1), _f32))
    return pl.pallas_call(_tcB_body, out_shape=out_shape)(
        accp, H, dinv_col, bc, ws)


def _score_top_body(k, saccp_ref, s_ref, dinv_ref, alive_ref, bs_ref,
                    scsel_ref, selv_ref):
    sacc = jnp.sum(saccp_ref[...], axis=0)[:N]
    dinv = dinv_ref[...][:N]
    pre = dinv * sacc + (dinv * dinv) * s_ref[...] + bs_ref[...]
    score = jnp.tanh(pre)
    alive = alive_ref[...]
    masked = jnp.where(alive > 0.0, score, -2.5)
    bits = lax.bitcast_convert_type(masked, _i32)
    key = jnp.where(bits < 0, bits ^ 0x7FFFFFFF, bits)

    def bis(_, lohi):
        lo, hi = lohi
        ulo = lo.astype(jnp.uint32)
        uhi = hi.astype(jnp.uint32)
        mid = (ulo + ((uhi - ulo) >> 1)).astype(_i32)
        cnt = jnp.sum((key > mid).astype(_i32))
        ge = cnt >= k
        return (jnp.where(ge, mid, lo), jnp.where(ge, hi, mid))
    lo0 = jnp.asarray(-(2 ** 31), _i32)
    hi0 = jnp.asarray(2 ** 31 - 1, _i32)
    lo, hi = lax.fori_loop(0, 32, bis, (lo0, hi0))

    c_hi = jnp.sum((key > hi).astype(_i32))
    need = k - c_hi
    tie = key == hi
    idx = lax.broadcasted_iota(_i32, (N,), 0)

    def bis2(_, lohi):
        lo2, hi2 = lohi
        mid = lo2 + (hi2 - lo2) // 2
        cnt = jnp.sum((tie & (idx <= mid)).astype(_i32))
        ge = cnt >= need
        return (jnp.where(ge, lo2, mid), jnp.where(ge, hi2, mid))
    lo2, hi2 = lax.fori_loop(0, 14, bis2,
                             (jnp.asarray(-1, _i32), jnp.asarray(N - 1, _i32)))

    sel = (key > hi) | (tie & (idx <= hi2) & (need > 0))
    selv = sel.astype(_f32)
    scsel_ref[...] = score * selv
    selv_ref[...] = selv


def _score_top_call(k, saccp, s_row, dinv_row, alive, bs):
    out_shape = (jax.ShapeDtypeStruct((N,), _f32),
                 jax.ShapeDtypeStruct((N,), _f32))
    body = functools.partial(_score_top_body, k)
    return pl.pallas_call(body, out_shape=out_shape)(
        saccp, s_row, dinv_row, alive, bs)


def _pool_body(k, H2_ref, scsel_ref, sel_ref, hn_ref, xr_ref):
    hn = H2_ref[...] * scsel_ref[...]
    hn_ref[...] = hn
    mx = jnp.max(jnp.where(sel_ref[...] > 0.0, hn, -3.0e38),
                 axis=0, keepdims=True)
    sm = jnp.sum(hn, axis=0, keepdims=True)
    xr_ref[...] = jnp.concatenate([mx, sm * (1.0 / k)], axis=1)


def _pool_call(k, H2, scsel_col, sel_col):
    out_shape = (jax.ShapeDtypeStruct((N, D), _f32),
                 jax.ShapeDtypeStruct((1, 2 * D), _f32))
    body = functools.partial(_pool_body, k)
    return pl.pallas_call(body, out_shape=out_shape)(H2, scsel_col, sel_col)


def _mlp_body(x1_ref, x2_ref, x3_ref, l1w_ref, l1b_ref, l2w_ref, l2b_ref,
              l3w_ref, l3b_ref, out_ref):
    z = x1_ref[...] + x2_ref[...] + x3_ref[...]
    z = jnp.maximum(
        jnp.dot(z, l1w_ref[...], preferred_element_type=_f32)
        + l1b_ref[...][None, :], 0.0)
    z = jnp.maximum(
        jnp.dot(z, l2w_ref[...], preferred_element_type=_f32)
        + l2b_ref[...][None, :], 0.0)
    z = jnp.dot(z, l3w_ref[...], preferred_element_type=_f32) \
        + l3b_ref[...][None, :]
    out_ref[...] = jax.nn.sigmoid(z)


def _mlp_call(x1, x2, x3, l1w, l1b, l2w, l2b, l3w, l3b):
    out_shape = jax.ShapeDtypeStruct((1, 1), _f32)
    return pl.pallas_call(_mlp_body, out_shape=out_shape)(
        x1, x2, x3, l1w, l1b, l2w, l2b, l3w, l3b)


def kernel(x, pos, edge_index, edge_attr, batch,
           conv1_W, conv1_b, pool1_W, pool1_b,
           conv2_W, conv2_b, pool2_W, pool2_b,
           conv3_W, conv3_b, pool3_W, pool3_b,
           lin1_W, lin1_b, lin2_W, lin2_b, lin3_W, lin3_b):
    src = jnp.zeros((EP,), _i32).at[:E].set(edge_index[0])
    dst = jnp.zeros((EP,), _i32).at[:E].set(edge_index[1])
    ew0 = jnp.zeros((EP,), _f32).at[:E].set(edge_attr)
    h = jnp.concatenate([x, pos], axis=1)
    alive = jnp.ones((N,), _f32)
    layers = ((conv1_W, conv1_b, pool1_W, pool1_b, 5000),
              (conv2_W, conv2_b, pool2_W, pool2_b, 2500),
              (conv3_W, conv3_b, pool3_W, pool3_b, 1250))
    xrs = []
    for (wc, bc, ws, bs, k) in layers:
        ew_i, degp = _edge_prep_call(src, dst, ew0, alive)
        dinv_row = _comb_deg_call(degp.reshape(NW, NP))
        dinv_col = dinv_row[:N].reshape(N, 1)
        H, hp = _tcA_call(h, wc, dinv_col)
        accp = _conv_call(hp, src, dst, ew_i).reshape(NC, NP, D)[:, :N]
        H2, sp, s = _tcB_call(accp, H, dinv_col, bc, ws)
        saccp = _score_call(sp.reshape(N), src, dst, ew_i)
        scsel, selv = _score_top_call(k, saccp.reshape(NW, NP),
                                      s.reshape(N), dinv_row, alive, bs)
        alive = selv
        h, xr = _pool_call(k, H2, scsel.reshape(N, 1), selv.reshape(N, 1))
        xrs.append(xr)
    return _mlp_call(xrs[0], xrs[1], xrs[2],
                     lin1_W, lin1_b, lin2_W, lin2_b, lin3_W, lin3_b)


# speedup vs baseline: 40.7746x; 1.9812x over previous
import functools

import jax
import jax.numpy as jnp
from jax import lax
from jax.experimental import pallas as pl
from jax.experimental.pallas import tpu as pltpu
from jax.experimental.pallas import tpu_sc as plsc

N = 10000
NP = 10240
E = 320000
D = 128
NC = 2
NS = 16
NW = NC * NS
EPT = 10240
EP = NW * EPT
CCH = 128
SCH = 2048
RPT = NP // NS

_f32 = jnp.float32
_i32 = jnp.int32


def _mesh():
    return plsc.VectorSubcoreMesh(core_axis_name="c", subcore_axis_name="s")


_sc_params = pltpu.CompilerParams(needs_layout_passes=False)


def _edge_prep_body(src_hbm, dst_hbm, ew_hbm, alive_hbm, ewo_hbm, degp_hbm,
                    alive_v, deg_v, src_v, dst_v, ew_v, ewo_v):
    c = lax.axis_index("c")
    s = lax.axis_index("s")
    wid = c * NS + s
    pltpu.sync_copy(alive_hbm, alive_v)
    zeros16 = jnp.zeros((16,), _f32)

    def zbody(i, carry):
        deg_v[pl.ds(pl.multiple_of(i * 16, 16), 16)] = zeros16
        return carry
    lax.fori_loop(0, NP // 16, zbody, 0)

    def cbody(ci, carry):
        off = wid * EPT + ci * SCH
        pltpu.sync_copy(src_hbm.at[pl.ds(off, SCH)], src_v)
        pltpu.sync_copy(dst_hbm.at[pl.ds(off, SCH)], dst_v)
        pltpu.sync_copy(ew_hbm.at[pl.ds(off, SCH)], ew_v)

        def ibody(j, icarry):
            sl = pl.ds(pl.multiple_of(j * 16, 16), 16)
            s16 = src_v[sl]
            d16 = dst_v[sl]
            w16 = ew_v[sl]
            a_s = plsc.load_gather(alive_v, [s16])
            a_d = plsc.load_gather(alive_v, [d16])
            wv = w16 * a_s * a_d
            ewo_v[sl] = wv
            plsc.addupdate_scatter(deg_v, [d16], wv)
            return icarry
        lax.fori_loop(0, SCH // 16, ibody, 0)
        pltpu.sync_copy(ewo_v, ewo_hbm.at[pl.ds(off, SCH)])
        return carry
    lax.fori_loop(0, EPT // SCH, cbody, 0)
    pltpu.sync_copy(deg_v, degp_hbm.at[pl.ds(wid * NP, NP)])


def _edge_prep_call(src, dst, ew, alive):
    out_type = (jax.ShapeDtypeStruct((EP,), _f32),
                jax.ShapeDtypeStruct((NW * NP,), _f32))
    scratch = [pltpu.VMEM((N,), _f32), pltpu.VMEM((NP,), _f32),
               pltpu.VMEM((SCH,), _i32), pltpu.VMEM((SCH,), _i32),
               pltpu.VMEM((SCH,), _f32), pltpu.VMEM((SCH,), _f32)]
    fn = pl.kernel(_edge_prep_body, out_type=out_type, mesh=_mesh(),
                   scratch_types=scratch, name="edge_prep",
                   compiler_params=_sc_params)
    return fn(src, dst, ew, alive)


def _conv_body(hp_hbm, src_hbm, dst_hbm, ew_hbm, out_hbm,
               acc_sh, src0, dst0, ew0_v, rows0, src1, dst1, ew1_v, rows1,
               sg0, sg1, ss0, ss1):
    c = lax.axis_index("c")
    s = lax.axis_index("s")
    wid = c * NS + s
    zeros16 = jnp.zeros((16,), _f32)
    srcb = (src0, src1)
    dstb = (dst0, dst1)
    ewb = (ew0_v, ew1_v)
    rowsb = (rows0, rows1)
    sgb = (sg0, sg1)
    ssb = (ss0, ss1)

    def zr(r, carry):
        for j in range(8):
            rows0[r, pl.ds(j * 16, 16)] = zeros16
        return carry
    lax.fori_loop(0, CCH, zr, 0)

    base = s * RPT
    for i in range(RPT // CCH):
        pltpu.sync_copy(rows0, acc_sh.at[pl.ds(base + i * CCH, CCH)])
    plsc.subcore_barrier()

    ebase = wid * EPT
    nch = EPT // CCH

    def prefetch(ci, b):
        off = ebase + ci * CCH
        pltpu.sync_copy(src_hbm.at[pl.ds(off, CCH)], srcb[b])
        pltpu.sync_copy(dst_hbm.at[pl.ds(off, CCH)], dstb[b])
        pltpu.sync_copy(ew_hbm.at[pl.ds(off, CCH)], ewb[b])
        pltpu.async_copy(hp_hbm.at[srcb[b]], rowsb[b], sgb[b])

    prefetch(0, 0)

    def cbody(ci, carry):
        for b in (0, 1):

            @pl.when((ci % 2) == b)
            def _():
                rows_v = rowsb[b]
                ew_v = ewb[b]
                pltpu.make_async_copy(hp_hbm.at[srcb[b]], rows_v,
                                      sgb[b]).wait()

                @pl.when(ci + 1 < nch)
                def _():
                    nb = 1 - b

                    @pl.when(ci >= 1)
                    def _():
                        pltpu.make_async_copy(
                            rowsb[nb], acc_sh.at[dstb[nb]], ssb[nb]).wait()
                    prefetch(ci + 1, nb)

                def sbody(jj, icarry):
                    w16 = ew_v[pl.ds(pl.multiple_of(jj * 16, 16), 16)]
                    for t in range(16):
                        w = w16[t]
                        r = jj * 16 + t
                        for j in range(8):
                            sl = pl.ds(j * 16, 16)
                            rows_v[r, sl] = rows_v[r, sl] * w
                    return icarry
                lax.fori_loop(0, CCH // 16, sbody, 0)
                pltpu.async_copy(rows_v, acc_sh.at[dstb[b]], ssb[b],
                                 add=True)
        return carry
    lax.fori_loop(0, nch, cbody, 0)
    pltpu.make_async_copy(rows0, acc_sh.at[dst0], ss0).wait()
    pltpu.make_async_copy(rows1, acc_sh.at[dst1], ss1).wait()
    plsc.subcore_barrier()
    pltpu.sync_copy(acc_sh.at[pl.ds(s * RPT, RPT)],
                    out_hbm.at[pl.ds(c * NP + s * RPT, RPT)])


def _conv_call(hp, src, dst, ew):
    out_type = jax.ShapeDtypeStruct((NC * NP, D), _f32)
    buf = [pltpu.VMEM((CCH,), _i32), pltpu.VMEM((CCH,), _i32),
           pltpu.VMEM((CCH,), _f32), pltpu.VMEM((CCH, D), _f32)]
    scratch = [pltpu.VMEM_SHARED((NP, D), _f32)] + buf + buf + \
        [pltpu.SemaphoreType.DMA] * 4
    fn = pl.kernel(_conv_body, out_type=out_type, mesh=_mesh(),
                   scratch_types=scratch, name="conv_pass",
                   compiler_params=_sc_params)
    return fn(hp, src, dst, ew)


def _score_body(sp_hbm, src_hbm, dst_hbm, ew_hbm, saccp_hbm,
                sp_v, sacc_v, src_v, dst_v, ew_v):
    c = lax.axis_index("c")
    s = lax.axis_index("s")
    wid = c * NS + s
    pltpu.sync_copy(sp_hbm, sp_v)
    zeros16 = jnp.zeros((16,), _f32)

    def zbody(i, carry):
        sacc_v[pl.ds(pl.multiple_of(i * 16, 16), 16)] = zeros16
        return carry
    lax.fori_loop(0, NP // 16, zbody, 0)

    def cbody(ci, carry):
        off = wid * EPT + ci * SCH
        pltpu.sync_copy(src_hbm.at[pl.ds(off, SCH)], src_v)
        pltpu.sync_copy(dst_hbm.at[pl.ds(off, SCH)], dst_v)
        pltpu.sync_copy(ew_hbm.at[pl.ds(off, SCH)], ew_v)

        def ibody(j, icarry):
            sl = pl.ds(pl.multiple_of(j * 16, 16), 16)
            s16 = src_v[sl]
            d16 = dst_v[sl]
            w16 = ew_v[sl]
            v = plsc.load_gather(sp_v, [s16]) * w16
            plsc.addupdate_scatter(sacc_v, [d16], v)
            return icarry
        lax.fori_loop(0, SCH // 16, ibody, 0)
        return carry
    lax.fori_loop(0, EPT // SCH, cbody, 0)
    pltpu.sync_copy(sacc_v, saccp_hbm.at[pl.ds(wid * NP, NP)])


def _score_call(sp, src, dst, ew):
    out_type = jax.ShapeDtypeStruct((NW * NP,), _f32)
    scratch = [pltpu.VMEM((N,), _f32), pltpu.VMEM((NP,), _f32),
               pltpu.VMEM((SCH,), _i32), pltpu.VMEM((SCH,), _i32),
               pltpu.VMEM((SCH,), _f32)]
    fn = pl.kernel(_score_body, out_type=out_type, mesh=_mesh(),
                   scratch_types=scratch, name="score_pass",
                   compiler_params=_sc_params)
    return fn(sp, src, dst, ew)


def _comb_deg_body(degp_ref, dinv_ref):
    deg = 1.0 + jnp.sum(degp_ref[...], axis=0)
    dinv_ref[...] = lax.rsqrt(deg)


def _comb_deg_call(degp):
    out_shape = jax.ShapeDtypeStruct((NP,), _f32)
    return pl.pallas_call(_comb_deg_body, out_shape=out_shape)(degp)


def _tcA_body(h_ref, wc_ref, dinv_ref, H_ref, hp_ref):
    H = jnp.dot(h_ref[...], wc_ref[...], preferred_element_type=_f32)
    H_ref[...] = H
    hp_ref[...] = H * dinv_ref[...]


def _tcA_call(h, wc, dinv_col):
    out_shape = (jax.ShapeDtypeStruct((N, D), _f32),
                 jax.ShapeDtypeStruct((N, D), _f32))
    return pl.pallas_call(_tcA_body, out_shape=out_shape)(h, wc, dinv_col)


def _tcB_body(accp_ref, H_ref, dinv_ref, bc_ref, ws_ref,
              H2_ref, sp_ref, s_ref):
    acc = accp_ref[0] + accp_ref[1]
    dinv = dinv_ref[...]
    out = dinv * acc + (dinv * dinv) * H_ref[...] + bc_ref[...][None, :]
    H2 = jnp.maximum(out, 0.0)
    s = jnp.dot(H2, ws_ref[...], preferred_element_type=_f32)
    H2_ref[...] = H2
    s_ref[...] = s
    sp_ref[...] = s * dinv


def _tcB_call(accp, H, dinv_col, bc, ws):
    out_shape = (jax.ShapeDtypeStruct((N, D), _f32),
                 jax.ShapeDtypeStruct((N, 1), _f32),
                 jax.ShapeDtypeStruct((N, 1), _f32))
    return pl.pallas_call(_tcB_body, out_shape=out_shape)(
        accp, H, dinv_col, bc, ws)


def _score_top_body(k, saccp_ref, s_ref, dinv_ref, alive_ref, bs_ref,
                    scsel_ref, selv_ref):
    sacc = jnp.sum(saccp_ref[...], axis=0)[:N]
    dinv = dinv_ref[...][:N]
    pre = dinv * sacc + (dinv * dinv) * s_ref[...] + bs_ref[...]
    score = jnp.tanh(pre)
    alive = alive_ref[...]
    masked = jnp.where(alive > 0.0, score, -2.5)
    bits = lax.bitcast_convert_type(masked, _i32)
    key = jnp.where(bits < 0, bits ^ 0x7FFFFFFF, bits)

    def bis(_, lohi):
        lo, hi = lohi
        ulo = lo.astype(jnp.uint32)
        uhi = hi.astype(jnp.uint32)
        mid = (ulo + ((uhi - ulo) >> 1)).astype(_i32)
        cnt = jnp.sum((key > mid).astype(_i32))
        ge = cnt >= k
        return (jnp.where(ge, mid, lo), jnp.where(ge, hi, mid))
    lo0 = jnp.asarray(-(2 ** 31), _i32)
    hi0 = jnp.asarray(2 ** 31 - 1, _i32)
    lo, hi = lax.fori_loop(0, 32, bis, (lo0, hi0))

    c_hi = jnp.sum((key > hi).astype(_i32))
    need = k - c_hi
    tie = key == hi
    idx = lax.broadcasted_iota(_i32, (N,), 0)

    def bis2(_, lohi):
        lo2, hi2 = lohi
        mid = lo2 + (hi2 - lo2) // 2
        cnt = jnp.sum((tie & (idx <= mid)).astype(_i32))
        ge = cnt >= need
        return (jnp.where(ge, lo2, mid), jnp.where(ge, hi2, mid))
    lo2, hi2 = lax.fori_loop(0, 14, bis2,
                             (jnp.asarray(-1, _i32), jnp.asarray(N - 1, _i32)))

    sel = (key > hi) | (tie & (idx <= hi2) & (need > 0))
    selv = sel.astype(_f32)
    scsel_ref[...] = score * selv
    selv_ref[...] = selv


def _score_top_call(k, saccp, s_row, dinv_row, alive, bs):
    out_shape = (jax.ShapeDtypeStruct((N,), _f32),
                 jax.ShapeDtypeStruct((N,), _f32))
    body = functools.partial(_score_top_body, k)
    return pl.pallas_call(body, out_shape=out_shape)(
        saccp, s_row, dinv_row, alive, bs)


def _pool_body(k, H2_ref, scsel_ref, sel_ref, hn_ref, xr_ref):
    hn = H2_ref[...] * scsel_ref[...]
    hn_ref[...] = hn
    mx = jnp.max(jnp.where(sel_ref[...] > 0.0, hn, -3.0e38),
                 axis=0, keepdims=True)
    sm = jnp.sum(hn, axis=0, keepdims=True)
    xr_ref[...] = jnp.concatenate([mx, sm * (1.0 / k)], axis=1)


def _pool_call(k, H2, scsel_col, sel_col):
    out_shape = (jax.ShapeDtypeStruct((N, D), _f32),
                 jax.ShapeDtypeStruct((1, 2 * D), _f32))
    body = functools.partial(_pool_body, k)
    return pl.pallas_call(body, out_shape=out_shape)(H2, scsel_col, sel_col)


def _mlp_body(x1_ref, x2_ref, x3_ref, l1w_ref, l1b_ref, l2w_ref, l2b_ref,
              l3w_ref, l3b_ref, out_ref):
    z = x1_ref[...] + x2_ref[...] + x3_ref[...]
    z = jnp.maximum(
        jnp.dot(z, l1w_ref[...], preferred_element_type=_f32)
        + l1b_ref[...][None, :], 0.0)
    z = jnp.maximum(
        jnp.dot(z, l2w_ref[...], preferred_element_type=_f32)
        + l2b_ref[...][None, :], 0.0)
    z = jnp.dot(z, l3w_ref[...], preferred_element_type=_f32) \
        + l3b_ref[...][None, :]
    out_ref[...] = jax.nn.sigmoid(z)


def _mlp_call(x1, x2, x3, l1w, l1b, l2w, l2b, l3w, l3b):
    out_shape = jax.ShapeDtypeStruct((1, 1), _f32)
    return pl.pallas_call(_mlp_body, out_shape=out_shape)(
        x1, x2, x3, l1w, l1b, l2w, l2b, l3w, l3b)


def kernel(x, pos, edge_index, edge_attr, batch,
           conv1_W, conv1_b, pool1_W, pool1_b,
           conv2_W, conv2_b, pool2_W, pool2_b,
           conv3_W, conv3_b, pool3_W, pool3_b,
           lin1_W, lin1_b, lin2_W, lin2_b, lin3_W, lin3_b):
    spread = (jnp.arange(EP, dtype=_i32) * 37) % N
    src = spread.at[:E].set(edge_index[0])
    dst = spread.at[:E].set(edge_index[1])
    ew0 = jnp.zeros((EP,), _f32).at[:E].set(edge_attr)
    h = jnp.concatenate([x, pos], axis=1)
    alive = jnp.ones((N,), _f32)
    layers = ((conv1_W, conv1_b, pool1_W, pool1_b, 5000),
              (conv2_W, conv2_b, pool2_W, pool2_b, 2500),
              (conv3_W, conv3_b, pool3_W, pool3_b, 1250))
    xrs = []
    for (wc, bc, ws, bs, k) in layers:
        ew_i, degp = _edge_prep_call(src, dst, ew0, alive)
        dinv_row = _comb_deg_call(degp.reshape(NW, NP))
        dinv_col = dinv_row[:N].reshape(N, 1)
        H, hp = _tcA_call(h, wc, dinv_col)
        accp = _conv_call(hp, src, dst, ew_i).reshape(NC, NP, D)[:, :N]
        H2, sp, s = _tcB_call(accp, H, dinv_col, bc, ws)
        saccp = _score_call(sp.reshape(N), src, dst, ew_i)
        scsel, selv = _score_top_call(k, saccp.reshape(NW, NP),
                                      s.reshape(N), dinv_row, alive, bs)
        alive = selv
        h, xr = _pool_call(k, H2, scsel.reshape(N, 1), selv.reshape(N, 1))
        xrs.append(xr)
    return _mlp_call(xrs[0], xrs[1], xrs[2],
                     lin1_W, lin1_b, lin2_W, lin2_b, lin3_W, lin3_b)
